# Initial kernel scaffold; baseline (speedup 1.0000x reference)
import jax
import jax.numpy as jnp
from jax.experimental import pallas as pl


def kernel(x, edge_index, W1, b1, g1, be1, W2, b2, g2, be2, W3, b3, g3, be3, W4, b4):
    raise NotImplementedError("write your pallas kernel here")


# speedup vs baseline: 26.2367x; 26.2367x over previous
import functools

import jax
import jax.numpy as jnp
from jax import lax
from jax.experimental import pallas as pl
from jax.experimental.pallas import tpu as pltpu
from jax.experimental.pallas import tpu_sc as plsc

_N = 50000
_NP = 50048
_E = 800000
_EROWS = 6256
_NT = 16
_NROW = _NP // _NT
_K = 8
_BN = 3128
_GRID = _NP // _BN

_f32 = jnp.float32


def _mesh():
    return plsc.VectorSubcoreMesh(core_axis_name="c", subcore_axis_name="s")


def _split8(total, s, nt=_NT):
    groups = total // 8
    base, rem = groups // nt, groups % nt
    r0 = 8 * (s * base + jnp.minimum(s, rem))
    cnt = 8 * jnp.where(s < rem, base + 1, base)
    return r0, cnt


def _deg(dst2d, ones8, zeros8):
    @functools.partial(
        pl.kernel,
        out_type=(jax.ShapeDtypeStruct((_NP, 8), _f32),
                  jax.ShapeDtypeStruct((_NP, 8), _f32)),
        mesh=_mesh(),
        compiler_params=pltpu.CompilerParams(use_tc_tiling_on_sc=False),
        scratch_types=[
            pltpu.VMEM((128, 8), _f32),
            pltpu.VMEM((_K, 128), jnp.int32),
            pltpu.VMEM_SHARED((_NP, 8), _f32),
        ],
    )
    def k(dst_hbm, ones_hbm, zeros_hbm, outa, outb, ones_v, didx, acc):
        c = lax.axis_index("c")
        s = lax.axis_index("s")
        nsl = pl.ds(s * _NROW, _NROW)
        pltpu.sync_copy(ones_hbm, ones_v)
        pltpu.sync_copy(zeros_hbm.at[nsl], acc.at[nsl])
        plsc.subcore_barrier()
        half = _EROWS // 2
        r0, cnt = _split8(half, s)
        r0 = r0 + c * half
        nblk = cnt // _K

        def blk(b, carry):
            base = r0 + b * _K
            pltpu.sync_copy(dst_hbm.at[pl.ds(base, _K)], didx)
            for j in range(_K):
                pltpu.sync_copy(ones_v, acc.at[didx.at[j]], add=True)
            return carry

        lax.fori_loop(0, nblk, blk, 0)
        plsc.subcore_barrier()

        @pl.when(c == 0)
        def _a():
            pltpu.sync_copy(acc.at[nsl], outa.at[nsl])

        @pl.when(c == 1)
        def _b():
            pltpu.sync_copy(acc.at[nsl], outb.at[nsl])

    return k(dst2d, ones8, zeros8)


def _prop8(u, src2d, dst2d, zeros8):
    @functools.partial(
        pl.kernel,
        out_type=(jax.ShapeDtypeStruct((_NP, 8), _f32),
                  jax.ShapeDtypeStruct((_NP, 8), _f32)),
        mesh=_mesh(),
        compiler_params=pltpu.CompilerParams(use_tc_tiling_on_sc=False),
        scratch_types=[
            pltpu.VMEM((_K, 128), jnp.int32),
            pltpu.VMEM((_K, 128), jnp.int32),
            pltpu.VMEM((_K, 128, 8), _f32),
            pltpu.VMEM_SHARED((_NP, 8), _f32),
            pltpu.SemaphoreType.DMA,
        ],
    )
    def k(u_hbm, src_hbm, dst_hbm, zeros_hbm, outa, outb,
          sidx, didx, rows, acc, sem):
        c = lax.axis_index("c")
        s = lax.axis_index("s")
        nsl = pl.ds(s * _NROW, _NROW)
        pltpu.sync_copy(zeros_hbm.at[nsl], acc.at[nsl])
        plsc.subcore_barrier()
        half = _EROWS // 2
        r0, cnt = _split8(half, s)
        r0 = r0 + c * half
        nblk = cnt // _K

        def blk(b, carry):
            base = r0 + b * _K
            pltpu.sync_copy(src_hbm.at[pl.ds(base, _K)], sidx)
            pltpu.sync_copy(dst_hbm.at[pl.ds(base, _K)], didx)
            descs = [pltpu.async_copy(u_hbm.at[sidx.at[j]], rows.at[j], sem)
                     for j in range(_K)]
            for d in descs:
                d.wait()
            for j in range(_K):
                pltpu.sync_copy(rows.at[j], acc.at[didx.at[j]], add=True)
            return carry

        lax.fori_loop(0, nblk, blk, 0)
        plsc.subcore_barrier()

        @pl.when(c == 0)
        def _a():
            pltpu.sync_copy(acc.at[nsl], outa.at[nsl])

        @pl.when(c == 1)
        def _b():
            pltpu.sync_copy(acc.at[nsl], outb.at[nsl])

    return k(u, src2d, dst2d, zeros8)


def _prop32(ua, ub, src2d, dst2d, zeros32):
    @functools.partial(
        pl.kernel,
        out_type=(jax.ShapeDtypeStruct((_NP, 32), _f32),
                  jax.ShapeDtypeStruct((_NP, 32), _f32)),
        mesh=_mesh(),
        compiler_params=pltpu.CompilerParams(use_tc_tiling_on_sc=False),
        scratch_types=[
            pltpu.VMEM((_K, 128), jnp.int32),
            pltpu.VMEM((_K, 128), jnp.int32),
            pltpu.VMEM((_K // 2, 128, 32), _f32),
            pltpu.VMEM_SHARED((_NP, 32), _f32),
            pltpu.SemaphoreType.DMA,
        ],
    )
    def k(ua_hbm, ub_hbm, src_hbm, dst_hbm, zeros_hbm, outa, outb,
          sidx, didx, rows, acc, sem):
        c = lax.axis_index("c")
        s = lax.axis_index("s")
        nsl = pl.ds(s * _NROW, _NROW)
        pltpu.sync_copy(zeros_hbm.at[nsl], acc.at[nsl])
        plsc.subcore_barrier()
        r0, cnt = _split8(_EROWS, s)
        nblk = cnt // _K

        def run(u_hbm):
            def blk(b, carry):
                base = r0 + b * _K
                pltpu.sync_copy(src_hbm.at[pl.ds(base, _K)], sidx)
                pltpu.sync_copy(dst_hbm.at[pl.ds(base, _K)], didx)
                for g in range(2):
                    descs = [pltpu.async_copy(u_hbm.at[sidx.at[4 * g + j]],
                                              rows.at[j], sem)
                             for j in range(4)]
                    for d in descs:
                        d.wait()
                    for j in range(4):
                        pltpu.sync_copy(rows.at[j], acc.at[didx.at[4 * g + j]],
                                        add=True)
                return carry

            lax.fori_loop(0, nblk, blk, 0)

        @pl.when(c == 0)
        def _ra():
            run(ua_hbm)

        @pl.when(c == 1)
        def _rb():
            run(ub_hbm)

        plsc.subcore_barrier()

        @pl.when(c == 0)
        def _a():
            pltpu.sync_copy(acc.at[nsl], outa.at[nsl])

        @pl.when(c == 1)
        def _b():
            pltpu.sync_copy(acc.at[nsl], outb.at[nsl])

    return k(ua, ub, src2d, dst2d, zeros32)


def _full(shape):
    return pl.BlockSpec(shape, lambda i: (0,) * len(shape))


def _rows(w):
    return pl.BlockSpec((_BN, w), lambda i: (i, 0))


def _k1_body(dega, degb, xp, dinv8, u0):
    deg = dega[...] + degb[...] + 1.0
    di = lax.rsqrt(deg)
    dinv8[...] = di
    u0[...] = di * xp[...]


def _k1(dega, degb, xp):
    return pl.pallas_call(
        _k1_body,
        grid=(_GRID,),
        in_specs=[_rows(8)] * 3,
        out_specs=[_rows(8)] * 2,
        out_shape=(jax.ShapeDtypeStruct((_NP, 8), _f32),) * 2,
    )(dega, degb, xp)


def _mm8_body(Sa, Sb, u0, dinv8, W, b, z, st):
    i = pl.program_id(0)
    p = dinv8[...] * (Sa[...] + Sb[...] + u0[...])
    zz = jnp.dot(p, W[...], preferred_element_type=_f32) + b[...]
    z[...] = zz

    @pl.when(i == 0)
    def _():
        st[...] = jnp.zeros_like(st)

    zm = jnp.where(_row_mask(i), zz, 0.0)
    st[0:1, :] += jnp.sum(zm, axis=0, keepdims=True)
    st[1:2, :] += jnp.sum(zm * zm, axis=0, keepdims=True)


def _mm8(Sa, Sb, u0, dinv8, W, b, wout):
    return pl.pallas_call(
        _mm8_body,
        grid=(_GRID,),
        in_specs=[_rows(8)] * 4 + [_full((8, wout)), _full((1, wout))],
        out_specs=[_rows(wout), _full((2, wout))],
        out_shape=(jax.ShapeDtypeStruct((_NP, wout), _f32),
                   jax.ShapeDtypeStruct((2, wout), _f32)),
    )(Sa, Sb, u0, dinv8, W, b)


def _mm64_body(Sa, Sb, ua, ub, dinv8, Wa, Wb, b, z, st):
    i = pl.program_id(0)
    di = dinv8[...][:, 0:1]
    pa = di * (Sa[...] + ua[...])
    pb = di * (Sb[...] + ub[...])
    zz = (jnp.dot(pa, Wa[...], preferred_element_type=_f32)
          + jnp.dot(pb, Wb[...], preferred_element_type=_f32) + b[...])
    z[...] = zz

    @pl.when(i == 0)
    def _():
        st[...] = jnp.zeros_like(st)

    zm = jnp.where(_row_mask(i), zz, 0.0)
    st[0:1, :] += jnp.sum(zm, axis=0, keepdims=True)
    st[1:2, :] += jnp.sum(zm * zm, axis=0, keepdims=True)


def _mm64(Sa, Sb, ua, ub, dinv8, Wa, Wb, b, wout):
    return pl.pallas_call(
        _mm64_body,
        grid=(_GRID,),
        in_specs=[_rows(32)] * 4 + [_rows(8),
                                    _full((32, wout)), _full((32, wout)),
                                    _full((1, wout))],
        out_specs=[_rows(wout), _full((2, wout))],
        out_shape=(jax.ShapeDtypeStruct((_NP, wout), _f32),
                   jax.ShapeDtypeStruct((2, wout), _f32)),
    )(Sa, Sb, ua, ub, dinv8, Wa, Wb, b)


def _bn_cols(z, st, g, be):
    m = st[0:1, :] * (1.0 / _N)
    v = st[1:2, :] * (1.0 / _N) - m * m
    h = (z - m) * lax.rsqrt(v + 1e-5) * g + be
    return jnp.where(h >= 0, h, 0.1 * h)


def _row_mask(i):
    gr = i * _BN + lax.broadcasted_iota(jnp.int32, (_BN, 1), 0)
    return gr < _N


def _bnact_body(z, st, g, be, dinv8, ua, ub):
    i = pl.program_id(0)
    h = _bn_cols(z[...], st[...], g[...], be[...])
    u = jnp.where(_row_mask(i), dinv8[...][:, 0:1] * h, 0.0)
    ua[...] = u[:, :32]
    ub[...] = u[:, 32:]


def _bnact(z, st, g, be, dinv8):
    return pl.pallas_call(
        _bnact_body,
        grid=(_GRID,),
        in_specs=[_rows(64), _full((2, 64)), _full((1, 64)), _full((1, 64)),
                  _rows(8)],
        out_specs=[_rows(32)] * 2,
        out_shape=(jax.ShapeDtypeStruct((_NP, 32), _f32),) * 2,
    )(z, st, g, be, dinv8)


def _k7_body(z, st, g, be, W4p, dinv8, u4):
    i = pl.program_id(0)
    h = _bn_cols(z[...], st[...], g[...], be[...])
    t = jnp.dot(h, W4p[...], preferred_element_type=_f32)
    u4[...] = jnp.where(_row_mask(i), dinv8[...] * t, 0.0)


def _k7(z, st, g, be, W4p, dinv8):
    return pl.pallas_call(
        _k7_body,
        grid=(_GRID,),
        in_specs=[_rows(256), _full((2, 256)), _full((1, 256)),
                  _full((1, 256)), _full((256, 8)), _rows(8)],
        out_specs=_rows(8),
        out_shape=jax.ShapeDtypeStruct((_NP, 8), _f32),
    )(z, st, g, be, W4p, dinv8)


def _k8_body(Sa, Sb, u4, dinv8, b4, y):
    r = dinv8[...] * (Sa[...] + Sb[...] + u4[...]) + b4[...]
    y[...] = jax.nn.sigmoid(r[:, 0:1])


def _k8(Sa, Sb, u4, dinv8, b4):
    return pl.pallas_call(
        _k8_body,
        grid=(_GRID,),
        in_specs=[_rows(8)] * 4 + [_full((1, 8))],
        out_specs=_rows(1),
        out_shape=jax.ShapeDtypeStruct((_NP, 1), _f32),
    )(Sa, Sb, u4, dinv8, b4)


def kernel(x, edge_index, W1, b1, g1, be1, W2, b2, g2, be2,
           W3, b3, g3, be3, W4, b4):
    npad = _EROWS * 128 - _E
    pad_idx = _N + (jnp.arange(npad, dtype=jnp.int32) % (_NP - _N))
    src2d = jnp.concatenate([edge_index[0], pad_idx]).reshape(_EROWS, 128)
    dst2d = jnp.concatenate([edge_index[1], pad_idx]).reshape(_EROWS, 128)
    xp = jnp.pad(x, ((0, _NP - _N), (0, 5)))
    W1p = jnp.pad(W1, ((0, 5), (0, 0)))
    W4p = jnp.pad(W4, ((0, 0), (0, 7)))
    zeros8 = jnp.zeros((_NP, 8), _f32)
    zeros32 = jnp.zeros((_NP, 32), _f32)
    ones8 = jnp.ones((128, 8), _f32)

    dega, degb = _deg(dst2d, ones8, zeros8)
    dinv8, u0 = _k1(dega, degb, xp)

    S0a, S0b = _prop8(u0, src2d, dst2d, zeros8)
    z1, st1 = _mm8(S0a, S0b, u0, dinv8, W1p, b1.reshape(1, -1), 64)
    u1a, u1b = _bnact(z1, st1, g1.reshape(1, -1), be1.reshape(1, -1), dinv8)

    S1a, S1b = _prop32(u1a, u1b, src2d, dst2d, zeros32)
    z2, st2 = _mm64(S1a, S1b, u1a, u1b, dinv8, W2[:32], W2[32:],
                    b2.reshape(1, -1), 64)
    u2a, u2b = _bnact(z2, st2, g2.reshape(1, -1), be2.reshape(1, -1), dinv8)

    S2a, S2b = _prop32(u2a, u2b, src2d, dst2d, zeros32)
    z3, st3 = _mm64(S2a, S2b, u2a, u2b, dinv8, W3[:32], W3[32:],
                    b3.reshape(1, -1), 256)
    u4 = _k7(z3, st3, g3.reshape(1, -1), be3.reshape(1, -1), W4p, dinv8)

    S4a, S4b = _prop8(u4, src2d, dst2d, zeros8)
    y = _k8(S4a, S4b, u4, dinv8, jnp.broadcast_to(b4.reshape(1, 1), (1, 8)))
    return y[:_N]


# speedup vs baseline: 27.5858x; 1.0514x over previous
import functools

import jax
import jax.numpy as jnp
from jax import lax
from jax.experimental import pallas as pl
from jax.experimental.pallas import tpu as pltpu
from jax.experimental.pallas import tpu_sc as plsc

_N = 50000
_NP = 50048
_E = 800000
_EROWS = 6256
_NT = 16
_NROW = _NP // _NT
_K = 8
_BN = 3128
_GRID = _NP // _BN

_f32 = jnp.float32


def _mesh():
    return plsc.VectorSubcoreMesh(core_axis_name="c", subcore_axis_name="s")


def _split8(total, s, nt=_NT):
    groups = total // 8
    base, rem = groups // nt, groups % nt
    r0 = 8 * (s * base + jnp.minimum(s, rem))
    cnt = 8 * jnp.where(s < rem, base + 1, base)
    return r0, cnt


def _zero_acc(zrow_hbm, zbuf, acc, s):
    base = s * _NROW
    pltpu.sync_copy(zrow_hbm, zbuf)
    nfull = _NROW // 128
    rem = _NROW - nfull * 128

    def zb(i, carry):
        pltpu.sync_copy(zbuf, acc.at[pl.ds(base + i * 128, 128)])
        return carry

    lax.fori_loop(0, nfull, zb, 0)
    pltpu.sync_copy(zbuf.at[pl.ds(0, rem)],
                    acc.at[pl.ds(base + nfull * 128, rem)])


def _prop_block(base, u_hbm, src_hbm, dst_hbm, sidx, didx,
                rA, rB, gsA, gsB, ssA, ssB, acc):
    pltpu.sync_copy(src_hbm.at[pl.ds(base, _K)], sidx)
    pltpu.sync_copy(dst_hbm.at[pl.ds(base, _K)], didx)

    def G(buf, gs, c0):
        return [pltpu.async_copy(u_hbm.at[sidx.at[c0 + j]], buf.at[j], gs)
                for j in range(2)]

    def S(buf, ss, c0):
        return [pltpu.async_copy(buf.at[j], acc.at[didx.at[c0 + j]], ss,
                                 add=True) for j in range(2)]

    def drain(ds_):
        for d in ds_:
            d.wait()

    gA = G(rA, gsA, 0)
    gB = G(rB, gsB, 2)
    drain(gA)
    sA = S(rA, ssA, 0)
    drain(gB)
    sB = S(rB, ssB, 2)
    drain(sA)
    gA = G(rA, gsA, 4)
    drain(sB)
    gB = G(rB, gsB, 6)
    drain(gA)
    sA = S(rA, ssA, 4)
    drain(gB)
    sB = S(rB, ssB, 6)
    drain(sA)
    drain(sB)


def _writeback(c, s, acc, outa, outb):
    nsl = pl.ds(s * _NROW, _NROW)

    @pl.when(c == 0)
    def _a():
        pltpu.sync_copy(acc.at[nsl], outa.at[nsl])

    @pl.when(c == 1)
    def _b():
        pltpu.sync_copy(acc.at[nsl], outb.at[nsl])


def _deg(dst2d, ones8, zrow8):
    @functools.partial(
        pl.kernel,
        out_type=(jax.ShapeDtypeStruct((_NP, 8), _f32),
                  jax.ShapeDtypeStruct((_NP, 8), _f32)),
        mesh=_mesh(),
        compiler_params=pltpu.CompilerParams(use_tc_tiling_on_sc=False),
        scratch_types=[
            pltpu.VMEM((128, 8), _f32),
            pltpu.VMEM((128, 8), _f32),
            pltpu.VMEM((_K, 128), jnp.int32),
            pltpu.VMEM_SHARED((_NP, 8), _f32),
            pltpu.SemaphoreType.DMA,
        ],
    )
    def k(dst_hbm, ones_hbm, zrow_hbm, outa, outb, ones_v, zbuf, didx,
          acc, ss):
        c = lax.axis_index("c")
        s = lax.axis_index("s")
        pltpu.sync_copy(ones_hbm, ones_v)
        _zero_acc(zrow_hbm, zbuf, acc, s)
        plsc.subcore_barrier()
        half = _EROWS // 2
        r0, cnt = _split8(half, s)
        r0 = r0 + c * half
        nblk = cnt // _K

        def blk(b, carry):
            base = r0 + b * _K
            pltpu.sync_copy(dst_hbm.at[pl.ds(base, _K)], didx)
            descs = [pltpu.async_copy(ones_v, acc.at[didx.at[j]], ss,
                                      add=True) for j in range(_K)]
            for d in descs:
                d.wait()
            return carry

        lax.fori_loop(0, nblk, blk, 0)
        plsc.subcore_barrier()
        _writeback(c, s, acc, outa, outb)

    return k(dst2d, ones8, zrow8)


def _prop8(u, src2d, dst2d, zrow8):
    @functools.partial(
        pl.kernel,
        out_type=(jax.ShapeDtypeStruct((_NP, 8), _f32),
                  jax.ShapeDtypeStruct((_NP, 8), _f32)),
        mesh=_mesh(),
        compiler_params=pltpu.CompilerParams(use_tc_tiling_on_sc=False),
        scratch_types=[
            pltpu.VMEM((_K, 128), jnp.int32),
            pltpu.VMEM((_K, 128), jnp.int32),
            pltpu.VMEM((2, 128, 8), _f32),
            pltpu.VMEM((2, 128, 8), _f32),
            pltpu.VMEM((128, 8), _f32),
            pltpu.VMEM_SHARED((_NP, 8), _f32),
            pltpu.SemaphoreType.DMA,
            pltpu.SemaphoreType.DMA,
            pltpu.SemaphoreType.DMA,
            pltpu.SemaphoreType.DMA,
        ],
    )
    def k(u_hbm, src_hbm, dst_hbm, zrow_hbm, outa, outb,
          sidx, didx, rA, rB, zbuf, acc, gsA, gsB, ssA, ssB):
        c = lax.axis_index("c")
        s = lax.axis_index("s")
        _zero_acc(zrow_hbm, zbuf, acc, s)
        plsc.subcore_barrier()
        half = _EROWS // 2
        r0, cnt = _split8(half, s)
        r0 = r0 + c * half
        nblk = cnt // _K

        def blk(b, carry):
            _prop_block(r0 + b * _K, u_hbm, src_hbm, dst_hbm, sidx, didx,
                        rA, rB, gsA, gsB, ssA, ssB, acc)
            return carry

        lax.fori_loop(0, nblk, blk, 0)
        plsc.subcore_barrier()
        _writeback(c, s, acc, outa, outb)

    return k(u, src2d, dst2d, zrow8)


def _prop32(ua, ub, src2d, dst2d, zrow32):
    @functools.partial(
        pl.kernel,
        out_type=(jax.ShapeDtypeStruct((_NP, 32), _f32),
                  jax.ShapeDtypeStruct((_NP, 32), _f32)),
        mesh=_mesh(),
        compiler_params=pltpu.CompilerParams(use_tc_tiling_on_sc=False),
        scratch_types=[
            pltpu.VMEM((_K, 128), jnp.int32),
            pltpu.VMEM((_K, 128), jnp.int32),
            pltpu.VMEM((2, 128, 32), _f32),
            pltpu.VMEM((2, 128, 32), _f32),
            pltpu.VMEM((128, 32), _f32),
            pltpu.VMEM_SHARED((_NP, 32), _f32),
            pltpu.SemaphoreType.DMA,
            pltpu.SemaphoreType.DMA,
            pltpu.SemaphoreType.DMA,
            pltpu.SemaphoreType.DMA,
        ],
    )
    def k(ua_hbm, ub_hbm, src_hbm, dst_hbm, zrow_hbm, outa, outb,
          sidx, didx, rA, rB, zbuf, acc, gsA, gsB, ssA, ssB):
        c = lax.axis_index("c")
        s = lax.axis_index("s")
        _zero_acc(zrow_hbm, zbuf, acc, s)
        plsc.subcore_barrier()
        r0, cnt = _split8(_EROWS, s)
        nblk = cnt // _K

        def run(u_hbm):
            def blk(b, carry):
                _prop_block(r0 + b * _K, u_hbm, src_hbm, dst_hbm, sidx, didx,
                            rA, rB, gsA, gsB, ssA, ssB, acc)
                return carry

            lax.fori_loop(0, nblk, blk, 0)

        @pl.when(c == 0)
        def _ra():
            run(ua_hbm)

        @pl.when(c == 1)
        def _rb():
            run(ub_hbm)

        plsc.subcore_barrier()
        _writeback(c, s, acc, outa, outb)

    return k(ua, ub, src2d, dst2d, zrow32)


def _full(shape):
    return pl.BlockSpec(shape, lambda i: (0,) * len(shape))


def _rows(w):
    return pl.BlockSpec((_BN, w), lambda i: (i, 0))


def _row_mask(i):
    gr = i * _BN + lax.broadcasted_iota(jnp.int32, (_BN, 1), 0)
    return gr < _N


def _k1_body(dega, degb, xp, dinv8, u0):
    deg = dega[...] + degb[...] + 1.0
    di = lax.rsqrt(deg)
    dinv8[...] = di
    u0[...] = di * xp[...]


def _k1(dega, degb, xp):
    return pl.pallas_call(
        _k1_body,
        grid=(_GRID,),
        in_specs=[_rows(8)] * 3,
        out_specs=[_rows(8)] * 2,
        out_shape=(jax.ShapeDtypeStruct((_NP, 8), _f32),) * 2,
    )(dega, degb, xp)


def _mm8_body(Sa, Sb, u0, dinv8, W, b, z, st):
    i = pl.program_id(0)
    p = dinv8[...] * (Sa[...] + Sb[...] + u0[...])
    zz = jnp.dot(p, W[...], preferred_element_type=_f32) + b[...]
    z[...] = zz

    @pl.when(i == 0)
    def _():
        st[...] = jnp.zeros_like(st)

    zm = jnp.where(_row_mask(i), zz, 0.0)
    st[0:1, :] += jnp.sum(zm, axis=0, keepdims=True)
    st[1:2, :] += jnp.sum(zm * zm, axis=0, keepdims=True)


def _mm8(Sa, Sb, u0, dinv8, W, b, wout):
    return pl.pallas_call(
        _mm8_body,
        grid=(_GRID,),
        in_specs=[_rows(8)] * 4 + [_full((8, wout)), _full((1, wout))],
        out_specs=[_rows(wout), _full((2, wout))],
        out_shape=(jax.ShapeDtypeStruct((_NP, wout), _f32),
                   jax.ShapeDtypeStruct((2, wout), _f32)),
    )(Sa, Sb, u0, dinv8, W, b)


def _mm64_body(Sa, Sb, ua, ub, dinv8, Wa, Wb, b, z, st):
    i = pl.program_id(0)
    di = dinv8[...][:, 0:1]
    pa = di * (Sa[...] + ua[...])
    pb = di * (Sb[...] + ub[...])
    zz = (jnp.dot(pa, Wa[...], preferred_element_type=_f32)
          + jnp.dot(pb, Wb[...], preferred_element_type=_f32) + b[...])
    z[...] = zz

    @pl.when(i == 0)
    def _():
        st[...] = jnp.zeros_like(st)

    zm = jnp.where(_row_mask(i), zz, 0.0)
    st[0:1, :] += jnp.sum(zm, axis=0, keepdims=True)
    st[1:2, :] += jnp.sum(zm * zm, axis=0, keepdims=True)


def _mm64(Sa, Sb, ua, ub, dinv8, Wa, Wb, b, wout):
    return pl.pallas_call(
        _mm64_body,
        grid=(_GRID,),
        in_specs=[_rows(32)] * 4 + [_rows(8),
                                    _full((32, wout)), _full((32, wout)),
                                    _full((1, wout))],
        out_specs=[_rows(wout), _full((2, wout))],
        out_shape=(jax.ShapeDtypeStruct((_NP, wout), _f32),
                   jax.ShapeDtypeStruct((2, wout), _f32)),
    )(Sa, Sb, ua, ub, dinv8, Wa, Wb, b)


def _bn_cols(z, st, g, be):
    m = st[0:1, :] * (1.0 / _N)
    v = st[1:2, :] * (1.0 / _N) - m * m
    h = (z - m) * lax.rsqrt(v + 1e-5) * g + be
    return jnp.where(h >= 0, h, 0.1 * h)


def _bnact_body(z, st, g, be, dinv8, ua, ub):
    i = pl.program_id(0)
    h = _bn_cols(z[...], st[...], g[...], be[...])
    u = jnp.where(_row_mask(i), dinv8[...][:, 0:1] * h, 0.0)
    ua[...] = u[:, :32]
    ub[...] = u[:, 32:]


def _bnact(z, st, g, be, dinv8):
    return pl.pallas_call(
        _bnact_body,
        grid=(_GRID,),
        in_specs=[_rows(64), _full((2, 64)), _full((1, 64)), _full((1, 64)),
                  _rows(8)],
        out_specs=[_rows(32)] * 2,
        out_shape=(jax.ShapeDtypeStruct((_NP, 32), _f32),) * 2,
    )(z, st, g, be, dinv8)


def _k7_body(z, st, g, be, W4p, dinv8, u4):
    i = pl.program_id(0)
    h = _bn_cols(z[...], st[...], g[...], be[...])
    t = jnp.dot(h, W4p[...], preferred_element_type=_f32)
    u4[...] = jnp.where(_row_mask(i), dinv8[...] * t, 0.0)


def _k7(z, st, g, be, W4p, dinv8):
    return pl.pallas_call(
        _k7_body,
        grid=(_GRID,),
        in_specs=[_rows(256), _full((2, 256)), _full((1, 256)),
                  _full((1, 256)), _full((256, 8)), _rows(8)],
        out_specs=_rows(8),
        out_shape=jax.ShapeDtypeStruct((_NP, 8), _f32),
    )(z, st, g, be, W4p, dinv8)


def _k8_body(Sa, Sb, u4, dinv8, b4, y):
    r = dinv8[...] * (Sa[...] + Sb[...] + u4[...]) + b4[...]
    y[...] = jax.nn.sigmoid(r[:, 0:1])


def _k8(Sa, Sb, u4, dinv8, b4):
    return pl.pallas_call(
        _k8_body,
        grid=(_GRID,),
        in_specs=[_rows(8)] * 4 + [_full((1, 8))],
        out_specs=_rows(1),
        out_shape=jax.ShapeDtypeStruct((_NP, 1), _f32),
    )(Sa, Sb, u4, dinv8, b4)


def kernel(x, edge_index, W1, b1, g1, be1, W2, b2, g2, be2,
           W3, b3, g3, be3, W4, b4):
    npad = _EROWS * 128 - _E
    pad_idx = _N + (jnp.arange(npad, dtype=jnp.int32) % (_NP - _N))
    src2d = jnp.concatenate([edge_index[0], pad_idx]).reshape(_EROWS, 128)
    dst2d = jnp.concatenate([edge_index[1], pad_idx]).reshape(_EROWS, 128)
    xp = jnp.pad(x, ((0, _NP - _N), (0, 5)))
    W1p = jnp.pad(W1, ((0, 5), (0, 0)))
    W4p = jnp.pad(W4, ((0, 0), (0, 7)))
    zrow8 = jnp.zeros((128, 8), _f32)
    zrow32 = jnp.zeros((128, 32), _f32)
    ones8 = jnp.ones((128, 8), _f32)

    dega, degb = _deg(dst2d, ones8, zrow8)
    dinv8, u0 = _k1(dega, degb, xp)

    S0a, S0b = _prop8(u0, src2d, dst2d, zrow8)
    z1, st1 = _mm8(S0a, S0b, u0, dinv8, W1p, b1.reshape(1, -1), 64)
    u1a, u1b = _bnact(z1, st1, g1.reshape(1, -1), be1.reshape(1, -1), dinv8)

    S1a, S1b = _prop32(u1a, u1b, src2d, dst2d, zrow32)
    z2, st2 = _mm64(S1a, S1b, u1a, u1b, dinv8, W2[:32], W2[32:],
                    b2.reshape(1, -1), 64)
    u2a, u2b = _bnact(z2, st2, g2.reshape(1, -1), be2.reshape(1, -1), dinv8)

    S2a, S2b = _prop32(u2a, u2b, src2d, dst2d, zrow32)
    z3, st3 = _mm64(S2a, S2b, u2a, u2b, dinv8, W3[:32], W3[32:],
                    b3.reshape(1, -1), 256)
    u4 = _k7(z3, st3, g3.reshape(1, -1), be3.reshape(1, -1), W4p, dinv8)

    S4a, S4b = _prop8(u4, src2d, dst2d, zrow8)
    y = _k8(S4a, S4b, u4, dinv8, jnp.broadcast_to(b4.reshape(1, 1), (1, 8)))
    return y[:_N]


# speedup vs baseline: 38.1682x; 1.3836x over previous
import functools

import jax
import jax.numpy as jnp
from jax import lax
from jax.experimental import pallas as pl
from jax.experimental.pallas import tpu as pltpu
from jax.experimental.pallas import tpu_sc as plsc

_N = 50000
_NP = 51200
_E = 800000
_EROWS = 6256
_NT = 16
_NROW = _NP // _NT
_K = 8

_f32 = jnp.float32


def _mesh():
    return plsc.VectorSubcoreMesh(core_axis_name="c", subcore_axis_name="s")


def _split8(total, s, nt=_NT):
    groups = total // 8
    base, rem = groups // nt, groups % nt
    r0 = 8 * (s * base + jnp.minimum(s, rem))
    cnt = 8 * jnp.where(s < rem, base + 1, base)
    return r0, cnt


def _zero_acc(zrow_hbm, zbuf, acc, s):
    base = s * _NROW
    pltpu.sync_copy(zrow_hbm, zbuf)
    nfull = _NROW // 128

    def zb(i, carry):
        pltpu.sync_copy(zbuf, acc.at[pl.ds(base + i * 128, 128)])
        return carry

    lax.fori_loop(0, nfull, zb, 0)


def _prop_block(base, u_hbm, src_hbm, dst_hbm, sidx, didx,
                rA, rB, gsA, gsB, ssA, ssB, acc):
    pltpu.sync_copy(src_hbm.at[pl.ds(base, _K)], sidx)
    pltpu.sync_copy(dst_hbm.at[pl.ds(base, _K)], didx)

    def G(buf, gs, c0):
        return [pltpu.async_copy(u_hbm.at[sidx.at[c0 + j]], buf.at[j], gs)
                for j in range(2)]

    def S(buf, ss, c0):
        return [pltpu.async_copy(buf.at[j], acc.at[didx.at[c0 + j]], ss,
                                 add=True) for j in range(2)]

    def drain(ds_):
        for d in ds_:
            d.wait()

    gA = G(rA, gsA, 0)
    gB = G(rB, gsB, 2)
    drain(gA)
    sA = S(rA, ssA, 0)
    drain(gB)
    sB = S(rB, ssB, 2)
    drain(sA)
    gA = G(rA, gsA, 4)
    drain(sB)
    gB = G(rB, gsB, 6)
    drain(gA)
    sA = S(rA, ssA, 4)
    drain(gB)
    sB = S(rB, ssB, 6)
    drain(sA)
    drain(sB)


def _writeback(c, s, acc, outa, outb):
    nsl = pl.ds(s * _NROW, _NROW)

    @pl.when(c == 0)
    def _a():
        pltpu.sync_copy(acc.at[nsl], outa.at[nsl])

    @pl.when(c == 1)
    def _b():
        pltpu.sync_copy(acc.at[nsl], outb.at[nsl])


def _deg(dst2d, ones8, zrow8):
    @functools.partial(
        pl.kernel,
        out_type=(jax.ShapeDtypeStruct((_NP, 8), _f32),
                  jax.ShapeDtypeStruct((_NP, 8), _f32)),
        mesh=_mesh(),
        compiler_params=pltpu.CompilerParams(use_tc_tiling_on_sc=False),
        scratch_types=[
            pltpu.VMEM((128, 8), _f32),
            pltpu.VMEM((128, 8), _f32),
            pltpu.VMEM((_K, 128), jnp.int32),
            pltpu.VMEM_SHARED((_NP, 8), _f32),
            pltpu.SemaphoreType.DMA,
        ],
    )
    def k(dst_hbm, ones_hbm, zrow_hbm, outa, outb, ones_v, zbuf, didx,
          acc, ss):
        c = lax.axis_index("c")
        s = lax.axis_index("s")
        pltpu.sync_copy(ones_hbm, ones_v)
        _zero_acc(zrow_hbm, zbuf, acc, s)
        plsc.subcore_barrier()
        half = _EROWS // 2
        r0, cnt = _split8(half, s)
        r0 = r0 + c * half
        nblk = cnt // _K

        def blk(b, carry):
            base = r0 + b * _K
            pltpu.sync_copy(dst_hbm.at[pl.ds(base, _K)], didx)
            descs = [pltpu.async_copy(ones_v, acc.at[didx.at[j]], ss,
                                      add=True) for j in range(_K)]
            for d in descs:
                d.wait()
            return carry

        lax.fori_loop(0, nblk, blk, 0)
        plsc.subcore_barrier()
        _writeback(c, s, acc, outa, outb)

    return k(dst2d, ones8, zrow8)


def _prop8(u, src2d, dst2d, zrow8):
    @functools.partial(
        pl.kernel,
        out_type=(jax.ShapeDtypeStruct((_NP, 8), _f32),
                  jax.ShapeDtypeStruct((_NP, 8), _f32)),
        mesh=_mesh(),
        compiler_params=pltpu.CompilerParams(use_tc_tiling_on_sc=False),
        scratch_types=[
            pltpu.VMEM((_K, 128), jnp.int32),
            pltpu.VMEM((_K, 128), jnp.int32),
            pltpu.VMEM((2, 128, 8), _f32),
            pltpu.VMEM((2, 128, 8), _f32),
            pltpu.VMEM((128, 8), _f32),
            pltpu.VMEM_SHARED((_NP, 8), _f32),
            pltpu.SemaphoreType.DMA,
            pltpu.SemaphoreType.DMA,
            pltpu.SemaphoreType.DMA,
            pltpu.SemaphoreType.DMA,
        ],
    )
    def k(u_hbm, src_hbm, dst_hbm, zrow_hbm, outa, outb,
          sidx, didx, rA, rB, zbuf, acc, gsA, gsB, ssA, ssB):
        c = lax.axis_index("c")
        s = lax.axis_index("s")
        _zero_acc(zrow_hbm, zbuf, acc, s)
        plsc.subcore_barrier()
        half = _EROWS // 2
        r0, cnt = _split8(half, s)
        r0 = r0 + c * half
        nblk = cnt // _K

        def blk(b, carry):
            _prop_block(r0 + b * _K, u_hbm, src_hbm, dst_hbm, sidx, didx,
                        rA, rB, gsA, gsB, ssA, ssB, acc)
            return carry

        lax.fori_loop(0, nblk, blk, 0)
        plsc.subcore_barrier()
        _writeback(c, s, acc, outa, outb)

    return k(u, src2d, dst2d, zrow8)


def _prop32(ua, ub, src2d, dst2d, zrow32):
    @functools.partial(
        pl.kernel,
        out_type=(jax.ShapeDtypeStruct((_NP, 32), _f32),
                  jax.ShapeDtypeStruct((_NP, 32), _f32)),
        mesh=_mesh(),
        compiler_params=pltpu.CompilerParams(use_tc_tiling_on_sc=False),
        scratch_types=[
            pltpu.VMEM((_K, 128), jnp.int32),
            pltpu.VMEM((_K, 128), jnp.int32),
            pltpu.VMEM((2, 128, 32), _f32),
            pltpu.VMEM((2, 128, 32), _f32),
            pltpu.VMEM((128, 32), _f32),
            pltpu.VMEM_SHARED((_NP, 32), _f32),
            pltpu.SemaphoreType.DMA,
            pltpu.SemaphoreType.DMA,
            pltpu.SemaphoreType.DMA,
            pltpu.SemaphoreType.DMA,
        ],
    )
    def k(ua_hbm, ub_hbm, src_hbm, dst_hbm, zrow_hbm, outa, outb,
          sidx, didx, rA, rB, zbuf, acc, gsA, gsB, ssA, ssB):
        c = lax.axis_index("c")
        s = lax.axis_index("s")
        _zero_acc(zrow_hbm, zbuf, acc, s)
        plsc.subcore_barrier()
        r0, cnt = _split8(_EROWS, s)
        nblk = cnt // _K

        def run(u_hbm):
            def blk(b, carry):
                _prop_block(r0 + b * _K, u_hbm, src_hbm, dst_hbm, sidx, didx,
                            rA, rB, gsA, gsB, ssA, ssB, acc)
                return carry

            lax.fori_loop(0, nblk, blk, 0)

        @pl.when(c == 0)
        def _ra():
            run(ua_hbm)

        @pl.when(c == 1)
        def _rb():
            run(ub_hbm)

        plsc.subcore_barrier()
        _writeback(c, s, acc, outa, outb)

    return k(ua, ub, src2d, dst2d, zrow32)


_PB = 200
_PN = _NP // 16
_GRID = _PN // _PB
_PREAL = _N // 16


def _pk(lanes):
    return pl.BlockSpec((_PB, lanes), lambda i: (i, 0))


def _full(shape):
    return pl.BlockSpec(shape, lambda i: (0,) * len(shape))


def _pmask(i):
    gr = i * _PB + lax.broadcasted_iota(jnp.int32, (_PB, 1), 0)
    return gr < _PREAL


def _mm_packed(q, W, win, wout):
    parts = [jnp.dot(q[:, k * win:(k + 1) * win], W,
                     preferred_element_type=_f32) for k in range(16)]
    return jnp.concatenate(parts, axis=1)


def _di_bcast(di8, wout):
    return jnp.concatenate(
        [jnp.broadcast_to(di8[:, 8 * k:8 * k + 1], (_PB, wout))
         for k in range(16)], axis=1)


def _tile16(v):
    return jnp.concatenate([v] * 16, axis=1)


def _fold16(s, w):
    out = s[:, 0:w]
    for k in range(1, 16):
        out = out + s[:, k * w:(k + 1) * w]
    return out


def _stats(st, i, zz, w):
    @pl.when(i == 0)
    def _():
        st[...] = jnp.zeros_like(st)

    zm = jnp.where(_pmask(i), zz, 0.0)
    st[0:1, :] += _fold16(jnp.sum(zm, axis=0, keepdims=True), w)
    st[1:2, :] += _fold16(jnp.sum(zm * zm, axis=0, keepdims=True), w)


def _k1_body(dega, degb, xp, dinvp, u0p):
    i = pl.program_id(0)
    deg = dega[...] + degb[...] + 1.0
    di = lax.rsqrt(deg)
    dinvp[...] = di
    u0p[...] = jnp.where(_pmask(i), di * xp[...], 0.0)


def _k1(degap, degbp, xp):
    return pl.pallas_call(
        _k1_body,
        grid=(_GRID,),
        in_specs=[_pk(128)] * 3,
        out_specs=[_pk(128)] * 2,
        out_shape=(jax.ShapeDtypeStruct((_PN, 128), _f32),) * 2,
    )(degap, degbp, xp)


def _mm8_body(Sa, Sb, u0, dinvp, W, bt, zp, st):
    i = pl.program_id(0)
    q = dinvp[...] * (Sa[...] + Sb[...] + u0[...])
    zz = _mm_packed(q, W[...], 8, 64) + bt[...]
    zp[...] = zz
    _stats(st, i, zz, 64)


def _mm8(Sap, Sbp, u0p, dinvp, W, bt):
    return pl.pallas_call(
        _mm8_body,
        grid=(_GRID,),
        in_specs=[_pk(128)] * 4 + [_full((8, 64)), _full((1, 1024))],
        out_specs=[_pk(1024), _full((2, 64))],
        out_shape=(jax.ShapeDtypeStruct((_PN, 1024), _f32),
                   jax.ShapeDtypeStruct((2, 64), _f32)),
    )(Sap, Sbp, u0p, dinvp, W, bt)


def _mm64_body(Sa, Sb, ua, ub, dinvp, Wa, Wb, bt, z, st, *, wout):
    i = pl.program_id(0)
    qa = Sa[...] + ua[...]
    qb = Sb[...] + ub[...]
    raw = (_mm_packed(qa, Wa[...], 32, wout)
           + _mm_packed(qb, Wb[...], 32, wout))
    zz = _di_bcast(dinvp[...], wout) * raw + bt[...]
    z[...] = zz
    _stats(st, i, zz, wout)


def _mm64(Sap, Sbp, uap, ubp, dinvp, Wa, Wb, bt, wout):
    return pl.pallas_call(
        functools.partial(_mm64_body, wout=wout),
        grid=(_GRID,),
        in_specs=[_pk(512)] * 4 + [_pk(128), _full((32, wout)),
                                   _full((32, wout)), _full((1, 16 * wout))],
        out_specs=[_pk(16 * wout), _full((2, wout))],
        out_shape=(jax.ShapeDtypeStruct((_PN, 16 * wout), _f32),
                   jax.ShapeDtypeStruct((2, wout), _f32)),
    )(Sap, Sbp, uap, ubp, dinvp, Wa, Wb, bt)


def _bn_packed(z, st, gt, bet):
    m = st[0:1, :] * (1.0 / _N)
    v = st[1:2, :] * (1.0 / _N) - m * m
    h = (z - _tile16(m)) * lax.rsqrt(_tile16(v) + 1e-5) * gt + bet
    return jnp.where(h >= 0, h, 0.1 * h)


def _bnact_body(zp, st, gt, bet, dinvp, uap, ubp):
    i = pl.program_id(0)
    h = _bn_packed(zp[...], st[...], gt[...], bet[...])
    u = jnp.where(_pmask(i), _di_bcast(dinvp[...], 64) * h, 0.0)
    uap[...] = jnp.concatenate(
        [u[:, 64 * k:64 * k + 32] for k in range(16)], axis=1)
    ubp[...] = jnp.concatenate(
        [u[:, 64 * k + 32:64 * (k + 1)] for k in range(16)], axis=1)


def _bnact(zp, st, gt, bet, dinvp):
    return pl.pallas_call(
        _bnact_body,
        grid=(_GRID,),
        in_specs=[_pk(1024), _full((2, 64)), _full((1, 1024)),
                  _full((1, 1024)), _pk(128)],
        out_specs=[_pk(512)] * 2,
        out_shape=(jax.ShapeDtypeStruct((_PN, 512), _f32),) * 2,
    )(zp, st, gt, bet, dinvp)


def _k7_body(zp, st, gt, bet, W4p, dinvp, u4p):
    i = pl.program_id(0)
    h = _bn_packed(zp[...], st[...], gt[...], bet[...])
    t = jnp.concatenate(
        [jnp.dot(h[:, 256 * k:256 * (k + 1)], W4p[...],
                 preferred_element_type=_f32) for k in range(16)], axis=1)
    u4p[...] = jnp.where(_pmask(i), dinvp[...] * t, 0.0)


def _k7(zp, st, gt, bet, W4p, dinvp):
    return pl.pallas_call(
        _k7_body,
        grid=(_GRID,),
        in_specs=[_pk(4096), _full((2, 256)), _full((1, 4096)),
                  _full((1, 4096)), _full((256, 8)), _pk(128)],
        out_specs=_pk(128),
        out_shape=jax.ShapeDtypeStruct((_PN, 128), _f32),
    )(zp, st, gt, bet, W4p, dinvp)


def _k8_body(Sa, Sb, u4, dinvp, b4t, yp):
    r = dinvp[...] * (Sa[...] + Sb[...] + u4[...]) + b4t[...]
    yp[...] = jax.nn.sigmoid(jnp.concatenate(
        [r[:, 8 * k:8 * k + 1] for k in range(16)], axis=1))


def _k8(Sap, Sbp, u4p, dinvp, b4t):
    return pl.pallas_call(
        _k8_body,
        grid=(_GRID,),
        in_specs=[_pk(128)] * 4 + [_full((1, 128))],
        out_specs=_pk(16),
        out_shape=jax.ShapeDtypeStruct((_PN, 16), _f32),
    )(Sap, Sbp, u4p, dinvp, b4t)


def _p8(a):
    return a.reshape(_PN, 128)


def _p32(a):
    return a.reshape(_PN, 512)


def kernel(x, edge_index, W1, b1, g1, be1, W2, b2, g2, be2,
           W3, b3, g3, be3, W4, b4):
    npad = _EROWS * 128 - _E
    pad_idx = _N + (jnp.arange(npad, dtype=jnp.int32) % (_NP - _N))
    src2d = jnp.concatenate([edge_index[0], pad_idx]).reshape(_EROWS, 128)
    dst2d = jnp.concatenate([edge_index[1], pad_idx]).reshape(_EROWS, 128)
    xp = jnp.pad(x, ((0, _NP - _N), (0, 5))).reshape(_PN, 128)
    W1p = jnp.pad(W1, ((0, 5), (0, 0)))
    W4p = jnp.pad(W4, ((0, 0), (0, 7)))
    zrow8 = jnp.zeros((128, 8), _f32)
    zrow32 = jnp.zeros((128, 32), _f32)
    ones8 = jnp.ones((128, 8), _f32)

    def t16(v):
        return jnp.tile(v.reshape(1, -1), (1, 16))

    dega, degb = _deg(dst2d, ones8, zrow8)
    dinvp, u0p = _k1(_p8(dega), _p8(degb), xp)

    S0a, S0b = _prop8(u0p.reshape(_NP, 8), src2d, dst2d, zrow8)
    z1p, st1 = _mm8(_p8(S0a), _p8(S0b), u0p, dinvp, W1p, t16(b1))
    u1ap, u1bp = _bnact(z1p, st1, t16(g1), t16(be1), dinvp)

    S1a, S1b = _prop32(u1ap.reshape(_NP, 32), u1bp.reshape(_NP, 32),
                       src2d, dst2d, zrow32)
    z2p, st2 = _mm64(_p32(S1a), _p32(S1b), u1ap, u1bp, dinvp,
                     W2[:32], W2[32:], t16(b2), 64)
    u2ap, u2bp = _bnact(z2p, st2, t16(g2), t16(be2), dinvp)

    S2a, S2b = _prop32(u2ap.reshape(_NP, 32), u2bp.reshape(_NP, 32),
                       src2d, dst2d, zrow32)
    z3p, st3 = _mm64(_p32(S2a), _p32(S2b), u2ap, u2bp, dinvp,
                     W3[:32], W3[32:], t16(b3), 256)
    u4p = _k7(z3p, st3, t16(g3), t16(be3), W4p, dinvp)

    S4a, S4b = _prop8(u4p.reshape(_NP, 8), src2d, dst2d, zrow8)
    yp = _k8(_p8(S4a), _p8(S4b), u4p, dinvp,
             jnp.broadcast_to(b4.reshape(1, 1), (1, 128)))
    return yp.reshape(_NP, 1)[:_N]
